# Initial kernel scaffold; baseline (speedup 1.0000x reference)
import jax
import jax.numpy as jnp
from jax.experimental import pallas as pl


def kernel(node_emb, batch, ln_g, ln_b, W1, b1, W2, b2, Ws, bs, Wc, bc):
    raise NotImplementedError("write your pallas kernel here")


# speedup vs baseline: 5.8000x; 5.8000x over previous
import functools

import jax
import jax.numpy as jnp
from jax import lax
from jax.experimental import pallas as pl
from jax.experimental.pallas import tpu as pltpu
from jax.experimental.pallas import tpu_sc as plsc

N = 320000
D = 128
G = 1024
EPS = 1e-5
NEG = -3.0e38

NC = 2
NS = 16
NW = NC * NS
RPW = N // NW
CHUNK = 400
NCHUNK = RPW // CHUNK
NL = D // 16


def _scalar_store(ref, idx, val):
  idx = jnp.asarray(idx, jnp.int32)
  base = (idx // 16) * 16
  lane = idx - base
  lanes = lax.broadcasted_iota(jnp.int32, (16,), 0)
  cur = ref[pl.ds(base, 16)]
  ref[pl.ds(base, 16)] = jnp.where(lanes == lane, jnp.full((16,), val), cur)


def _sc_reduce_build():
  mesh = plsc.VectorSubcoreMesh(core_axis_name="c", subcore_axis_name="s")
  out_type = (
      jax.ShapeDtypeStruct((G, D), jnp.float32),
      jax.ShapeDtypeStruct((G, D), jnp.float32),
      jax.ShapeDtypeStruct((NW, G), jnp.float32),
      jax.ShapeDtypeStruct((NW, 16), jnp.int32),
      jax.ShapeDtypeStruct((NW, 16), jnp.float32),
      jax.ShapeDtypeStruct((NW, 2 * D), jnp.float32),
      jax.ShapeDtypeStruct((NW, 2 * D), jnp.float32),
  )
  scratch = [
      pltpu.VMEM((CHUNK, D), jnp.float32),
      pltpu.VMEM((CHUNK + 16,), jnp.int32),
      pltpu.VMEM((D,), jnp.float32),
      pltpu.VMEM((D,), jnp.float32),
      pltpu.VMEM((G,), jnp.float32),
      pltpu.VMEM((2 * D,), jnp.float32),
      pltpu.VMEM((2 * D,), jnp.float32),
      pltpu.VMEM((16,), jnp.float32),
      pltpu.VMEM((16,), jnp.int32),
  ]

  @functools.partial(pl.kernel, out_type=out_type, mesh=mesh,
                     scratch_types=scratch)
  def sc_reduce(ne, bt, isum, imax, icnt, esid, ecnt, esum, emax,
                rowbuf, batbuf, fsum, fmax, cloc, es, em, ec, eid):
    cid = lax.axis_index("c")
    sid = lax.axis_index("s")
    wid = sid * NC + cid
    row0 = wid * RPW

    def zcnt(i, _):
      cloc[pl.ds(i * 16, 16)] = jnp.zeros((16,), jnp.float32)
      return 0
    lax.fori_loop(0, G // 16, zcnt, 0)
    eid[...] = jnp.full((16,), -1, jnp.int32)
    ec[...] = jnp.zeros((16,), jnp.float32)

    def row_body(i, carry):
      seg, cnt, nfl = carry[0], carry[1], carry[2]
      s = carry[3:3 + NL]
      m = carry[3 + NL:3 + 2 * NL]
      b = batbuf[pl.ds(i, 16)][0]
      rows = [rowbuf[i, pl.ds(j * 16, 16)] for j in range(NL)]
      boundary = b != seg
      do_flush = jnp.logical_and(boundary, cnt > 0)

      @pl.when(do_flush)
      def _():
        cf = cnt.astype(jnp.float32)

        @pl.when(nfl == 0)
        def _():
          for j in range(NL):
            es[pl.ds(j * 16, 16)] = s[j]
            em[pl.ds(j * 16, 16)] = m[j]
          _scalar_store(ec, 0, cf)
          _scalar_store(eid, 0, seg)

        @pl.when(nfl > 0)
        def _():
          for j in range(NL):
            fsum[pl.ds(j * 16, 16)] = s[j]
            fmax[pl.ds(j * 16, 16)] = m[j]
          pltpu.sync_copy(fsum, isum.at[seg])
          pltpu.sync_copy(fmax, imax.at[seg])
          _scalar_store(cloc, seg, cf)

      news = [jnp.where(boundary, rows[j], s[j] + rows[j]) for j in range(NL)]
      newm = [jnp.where(boundary, rows[j], jnp.maximum(m[j], rows[j]))
              for j in range(NL)]
      newseg = jnp.where(boundary, b, seg)
      newcnt = jnp.where(boundary, 1, cnt + 1)
      newnfl = jnp.where(do_flush, nfl + 1, nfl)
      return (newseg, newcnt, newnfl) + tuple(news) + tuple(newm)

    def chunk_body(c, carry):
      base = row0 + c * CHUNK
      pltpu.sync_copy(ne.at[pl.ds(base, CHUNK)], rowbuf)
      pltpu.sync_copy(bt.at[pl.ds(base, CHUNK)], batbuf.at[pl.ds(0, CHUNK)])
      return lax.fori_loop(0, CHUNK, row_body, carry)

    zero = jnp.zeros((16,), jnp.float32)
    init = (jnp.int32(-1), jnp.int32(0), jnp.int32(0)) + (zero,) * (2 * NL)
    carry = lax.fori_loop(0, NCHUNK, chunk_body, init)

    seg, cnt, nfl = carry[0], carry[1], carry[2]
    s = carry[3:3 + NL]
    m = carry[3 + NL:3 + 2 * NL]
    cf = cnt.astype(jnp.float32)

    @pl.when(nfl == 0)
    def _():
      for j in range(NL):
        es[pl.ds(j * 16, 16)] = s[j]
        em[pl.ds(j * 16, 16)] = m[j]
      _scalar_store(ec, 0, cf)
      _scalar_store(eid, 0, seg)

    @pl.when(nfl > 0)
    def _():
      for j in range(NL):
        es[pl.ds(D + j * 16, 16)] = s[j]
        em[pl.ds(D + j * 16, 16)] = m[j]
      _scalar_store(ec, 1, cf)
      _scalar_store(eid, 1, seg)

    pltpu.sync_copy(cloc, icnt.at[wid])
    pltpu.sync_copy(eid, esid.at[wid])
    pltpu.sync_copy(ec, ecnt.at[wid])
    pltpu.sync_copy(es, esum.at[wid])
    pltpu.sync_copy(em, emax.at[wid])

  return sc_reduce


_sc_reduce = _sc_reduce_build()


def _tc_head(isum, imax, icnt, esid2, esidv, ecnt2, esum2, emax2,
             ln_g, ln_b, W1, b1, W2, b2, Ws, bs, Wc, bc,
             score_out, cls_out, mx):
  dn = (((0,), (1,)), ((), ()))
  ones_w = jnp.ones((1, NW), jnp.float32)
  cnt_int = lax.dot_general(icnt[...], ones_w, dn,
                            preferred_element_type=jnp.float32)
  valid = cnt_int > 0.0

  mx[pl.ds(0, G), :] = jnp.where(valid, imax[...], NEG)
  mx[pl.ds(G, 8), :] = jnp.full((8, D), NEG, jnp.float32)

  def merge_max(r, _):
    sd = esid2[r]
    tid = jnp.where(sd >= 0, sd, G)
    row = emax2[pl.ds(r, 1), :]
    cur = mx[pl.ds(tid, 1), :]
    mx[pl.ds(tid, 1), :] = jnp.maximum(cur, row)
    return 0
  lax.fori_loop(0, 2 * NW, merge_max, 0)

  ids = jax.lax.broadcasted_iota(jnp.int32, (2 * NW, G), 1)
  oh = (esidv[...] == ids).astype(jnp.float32)
  dn0 = (((0,), (0,)), ((), ()))
  s = jnp.where(valid, isum[...], 0.0)
  s = s + lax.dot_general(oh, esum2[...], dn0,
                          preferred_element_type=jnp.float32)
  cnt = cnt_int + lax.dot_general(oh, ecnt2[...], dn0,
                                  preferred_element_type=jnp.float32)

  mean = s / jnp.maximum(cnt, 1.0)
  mfin = jnp.where(cnt > 0.0, mx[pl.ds(0, G), :], 0.0)

  g = jnp.concatenate([s, mean, mfin], axis=1)
  mu = jnp.mean(g, axis=1, keepdims=True)
  var = jnp.mean((g - mu) ** 2, axis=1, keepdims=True)
  h = (g - mu) * jax.lax.rsqrt(var + EPS) * ln_g[...] + ln_b[...]

  h = h @ W1[...] + b1[...]
  h = h * jax.nn.sigmoid(h)
  h = h @ W2[...] + b2[...]
  h = h * jax.nn.sigmoid(h)
  score_out[...] = jax.nn.sigmoid(h @ Ws[...] + bs[...])
  cls_out[...] = h @ Wc[...] + bc[...]


@jax.jit
def kernel(node_emb, batch, ln_g, ln_b, W1, b1, W2, b2, Ws, bs, Wc, bc):
  isum, imax, icnt, esid, ecnt, esum, emax = _sc_reduce(node_emb, batch)

  esid2 = esid[:, :2].reshape(2 * NW)
  ecnt2 = ecnt[:, :2].reshape(2 * NW, 1)
  esum2 = esum.reshape(2 * NW, D)
  emax2 = emax.reshape(2 * NW, D)

  vspec = pl.BlockSpec(memory_space=pltpu.VMEM)
  sspec = pl.BlockSpec(memory_space=pltpu.SMEM)
  score, cls = pl.pallas_call(
      _tc_head,
      out_shape=[jax.ShapeDtypeStruct((G, 1), jnp.float32),
                 jax.ShapeDtypeStruct((G, 4), jnp.float32)],
      in_specs=[vspec, vspec, vspec, sspec] + [vspec] * 14,
      out_specs=[vspec, vspec],
      scratch_shapes=[pltpu.VMEM((G + 8, D), jnp.float32)],
  )(isum, imax, icnt, esid2, esid2.reshape(2 * NW, 1), ecnt2, esum2, emax2,
    ln_g.reshape(1, 3 * D), ln_b.reshape(1, 3 * D), W1, b1.reshape(1, D),
    W2, b2.reshape(1, D // 2), Ws, bs.reshape(1, 1), Wc, bc.reshape(1, 4))
  return score[:, 0], cls


# speedup vs baseline: 7.3708x; 1.2708x over previous
import functools

import jax
import jax.numpy as jnp
from jax import lax
from jax.experimental import pallas as pl
from jax.experimental.pallas import tpu as pltpu
from jax.experimental.pallas import tpu_sc as plsc

N = 320000
D = 128
G = 1024
EPS = 1e-5
NEG = -3.0e38

NC = 2
NS = 16
NW = NC * NS
RPW = N // NW
CHUNK = 200
NCHUNK = RPW // CHUNK
NL = D // 16


def _scalar_store(ref, idx, val):
  idx = jnp.asarray(idx, jnp.int32)
  base = (idx // 16) * 16
  lane = idx - base
  lanes = lax.broadcasted_iota(jnp.int32, (16,), 0)
  cur = ref[pl.ds(base, 16)]
  ref[pl.ds(base, 16)] = jnp.where(lanes == lane, jnp.full((16,), val), cur)


def _sc_reduce_build():
  mesh = plsc.VectorSubcoreMesh(core_axis_name="c", subcore_axis_name="s")
  out_type = (
      jax.ShapeDtypeStruct((G, D), jnp.float32),
      jax.ShapeDtypeStruct((G, D), jnp.float32),
      jax.ShapeDtypeStruct((NW, G), jnp.float32),
      jax.ShapeDtypeStruct((NW, 16), jnp.int32),
      jax.ShapeDtypeStruct((NW, 16), jnp.float32),
      jax.ShapeDtypeStruct((NW, 2 * D), jnp.float32),
      jax.ShapeDtypeStruct((NW, 2 * D), jnp.float32),
  )
  scratch = [
      pltpu.VMEM((CHUNK, D), jnp.float32),
      pltpu.VMEM((CHUNK, D), jnp.float32),
      pltpu.VMEM((CHUNK + 16,), jnp.int32),
      pltpu.VMEM((CHUNK + 16,), jnp.int32),
      pltpu.VMEM((D,), jnp.float32),
      pltpu.VMEM((D,), jnp.float32),
      pltpu.VMEM((G,), jnp.float32),
      pltpu.VMEM((2 * D,), jnp.float32),
      pltpu.VMEM((2 * D,), jnp.float32),
      pltpu.VMEM((16,), jnp.float32),
      pltpu.VMEM((16,), jnp.int32),
      pltpu.SemaphoreType.DMA,
      pltpu.SemaphoreType.DMA,
      pltpu.SemaphoreType.DMA,
      pltpu.SemaphoreType.DMA,
  ]

  @functools.partial(pl.kernel, out_type=out_type, mesh=mesh,
                     scratch_types=scratch)
  def sc_reduce(ne, bt, isum, imax, icnt, esid, ecnt, esum, emax,
                rb0, rb1, bb0, bb1, fsum, fmax, cloc, es, em, ec, eid,
                smr0, smr1, smb0, smb1):
    cid = lax.axis_index("c")
    sid = lax.axis_index("s")
    wid = sid * NC + cid
    row0 = wid * RPW

    def start_chunk(c, rb, bb, smr, smb):
      base = row0 + c * CHUNK
      pltpu.make_async_copy(ne.at[pl.ds(base, CHUNK)], rb, smr).start()
      pltpu.make_async_copy(bt.at[pl.ds(base, CHUNK)],
                            bb.at[pl.ds(0, CHUNK)], smb).start()

    def wait_chunk(rb, bb, smr, smb):
      pltpu.make_async_copy(ne.at[pl.ds(0, CHUNK)], rb, smr).wait()
      pltpu.make_async_copy(bt.at[pl.ds(0, CHUNK)],
                            bb.at[pl.ds(0, CHUNK)], smb).wait()

    def zcnt(i, _):
      cloc[pl.ds(i * 16, 16)] = jnp.zeros((16,), jnp.float32)
      return 0
    lax.fori_loop(0, G // 16, zcnt, 0)
    eid[...] = jnp.full((16,), -1, jnp.int32)
    ec[...] = jnp.zeros((16,), jnp.float32)

    def make_row_body(rb, bb):
      def row_body(i, carry):
        seg, cnt, nfl = carry[0], carry[1], carry[2]
        s = carry[3:3 + NL]
        m = carry[3 + NL:3 + 2 * NL]
        b = bb[pl.ds(i, 16)][0]
        rows = [rb[i, pl.ds(j * 16, 16)] for j in range(NL)]
        boundary = b != seg
        do_flush = jnp.logical_and(boundary, cnt > 0)

        @pl.when(do_flush)
        def _():
          cf = cnt.astype(jnp.float32)

          @pl.when(nfl == 0)
          def _():
            for j in range(NL):
              es[pl.ds(j * 16, 16)] = s[j]
              em[pl.ds(j * 16, 16)] = m[j]
            _scalar_store(ec, 0, cf)
            _scalar_store(eid, 0, seg)

          @pl.when(nfl > 0)
          def _():
            for j in range(NL):
              fsum[pl.ds(j * 16, 16)] = s[j]
              fmax[pl.ds(j * 16, 16)] = m[j]
            pltpu.sync_copy(fsum, isum.at[seg])
            pltpu.sync_copy(fmax, imax.at[seg])
            _scalar_store(cloc, seg, cf)

        news = [jnp.where(boundary, rows[j], s[j] + rows[j])
                for j in range(NL)]
        newm = [jnp.where(boundary, rows[j], jnp.maximum(m[j], rows[j]))
                for j in range(NL)]
        newseg = jnp.where(boundary, b, seg)
        newcnt = jnp.where(boundary, 1, cnt + 1)
        newnfl = jnp.where(do_flush, nfl + 1, nfl)
        return (newseg, newcnt, newnfl) + tuple(news) + tuple(newm)
      return row_body

    row_body0 = make_row_body(rb0, bb0)
    row_body1 = make_row_body(rb1, bb1)

    def pair_body(i, carry):
      c0 = 2 * i
      start_chunk(c0 + 1, rb1, bb1, smr1, smb1)
      wait_chunk(rb0, bb0, smr0, smb0)
      carry = lax.fori_loop(0, CHUNK, row_body0, carry)

      @pl.when(i < NCHUNK // 2 - 1)
      def _():
        start_chunk(c0 + 2, rb0, bb0, smr0, smb0)

      wait_chunk(rb1, bb1, smr1, smb1)
      return lax.fori_loop(0, CHUNK, row_body1, carry)

    zero = jnp.zeros((16,), jnp.float32)
    init = (jnp.int32(-1), jnp.int32(0), jnp.int32(0)) + (zero,) * (2 * NL)
    start_chunk(0, rb0, bb0, smr0, smb0)
    carry = lax.fori_loop(0, NCHUNK // 2, pair_body, init)

    seg, cnt, nfl = carry[0], carry[1], carry[2]
    s = carry[3:3 + NL]
    m = carry[3 + NL:3 + 2 * NL]
    cf = cnt.astype(jnp.float32)

    @pl.when(nfl == 0)
    def _():
      for j in range(NL):
        es[pl.ds(j * 16, 16)] = s[j]
        em[pl.ds(j * 16, 16)] = m[j]
      _scalar_store(ec, 0, cf)
      _scalar_store(eid, 0, seg)

    @pl.when(nfl > 0)
    def _():
      for j in range(NL):
        es[pl.ds(D + j * 16, 16)] = s[j]
        em[pl.ds(D + j * 16, 16)] = m[j]
      _scalar_store(ec, 1, cf)
      _scalar_store(eid, 1, seg)

    pltpu.sync_copy(cloc, icnt.at[wid])
    pltpu.sync_copy(eid, esid.at[wid])
    pltpu.sync_copy(ec, ecnt.at[wid])
    pltpu.sync_copy(es, esum.at[wid])
    pltpu.sync_copy(em, emax.at[wid])

  return sc_reduce


_sc_reduce = _sc_reduce_build()


def _tc_head(isum, imax, icnt, esid2, esidv, ecnt2, esum2, emax2,
             ln_g, ln_b, W1, b1, W2, b2, Ws, bs, Wc, bc,
             score_out, cls_out, mx):
  dn = (((0,), (1,)), ((), ()))
  ones_w = jnp.ones((1, NW), jnp.float32)
  cnt_int = lax.dot_general(icnt[...], ones_w, dn,
                            preferred_element_type=jnp.float32)
  valid = cnt_int > 0.0

  mx[pl.ds(0, G), :] = jnp.where(valid, imax[...], NEG)
  mx[pl.ds(G, 8), :] = jnp.full((8, D), NEG, jnp.float32)

  def merge_max(r, _):
    sd = esid2[r]
    tid = jnp.where(sd >= 0, sd, G)
    row = emax2[pl.ds(r, 1), :]
    cur = mx[pl.ds(tid, 1), :]
    mx[pl.ds(tid, 1), :] = jnp.maximum(cur, row)
    return 0
  lax.fori_loop(0, 2 * NW, merge_max, 0)

  ids = jax.lax.broadcasted_iota(jnp.int32, (2 * NW, G), 1)
  oh = (esidv[...] == ids).astype(jnp.float32)
  dn0 = (((0,), (0,)), ((), ()))
  s = jnp.where(valid, isum[...], 0.0)
  s = s + lax.dot_general(oh, esum2[...], dn0,
                          preferred_element_type=jnp.float32)
  cnt = cnt_int + lax.dot_general(oh, ecnt2[...], dn0,
                                  preferred_element_type=jnp.float32)

  mean = s / jnp.maximum(cnt, 1.0)
  mfin = jnp.where(cnt > 0.0, mx[pl.ds(0, G), :], 0.0)

  g = jnp.concatenate([s, mean, mfin], axis=1)
  mu = jnp.mean(g, axis=1, keepdims=True)
  var = jnp.mean((g - mu) ** 2, axis=1, keepdims=True)
  h = (g - mu) * jax.lax.rsqrt(var + EPS) * ln_g[...] + ln_b[...]

  h = h @ W1[...] + b1[...]
  h = h * jax.nn.sigmoid(h)
  h = h @ W2[...] + b2[...]
  h = h * jax.nn.sigmoid(h)
  score_out[...] = jax.nn.sigmoid(h @ Ws[...] + bs[...])
  cls_out[...] = h @ Wc[...] + bc[...]


@jax.jit
def kernel(node_emb, batch, ln_g, ln_b, W1, b1, W2, b2, Ws, bs, Wc, bc):
  isum, imax, icnt, esid, ecnt, esum, emax = _sc_reduce(node_emb, batch)

  esid2 = esid[:, :2].reshape(2 * NW)
  ecnt2 = ecnt[:, :2].reshape(2 * NW, 1)
  esum2 = esum.reshape(2 * NW, D)
  emax2 = emax.reshape(2 * NW, D)

  vspec = pl.BlockSpec(memory_space=pltpu.VMEM)
  sspec = pl.BlockSpec(memory_space=pltpu.SMEM)
  score, cls = pl.pallas_call(
      _tc_head,
      out_shape=[jax.ShapeDtypeStruct((G, 1), jnp.float32),
                 jax.ShapeDtypeStruct((G, 4), jnp.float32)],
      in_specs=[vspec, vspec, vspec, sspec] + [vspec] * 14,
      out_specs=[vspec, vspec],
      scratch_shapes=[pltpu.VMEM((G + 8, D), jnp.float32)],
  )(isum, imax, icnt, esid2, esid2.reshape(2 * NW, 1), ecnt2, esum2, emax2,
    ln_g.reshape(1, 3 * D), ln_b.reshape(1, 3 * D), W1, b1.reshape(1, D),
    W2, b2.reshape(1, D // 2), Ws, bs.reshape(1, 1), Wc, bc.reshape(1, 4))
  return score[:, 0], cls


# speedup vs baseline: 13.6770x; 1.8556x over previous
import functools

import jax
import jax.numpy as jnp
from jax import lax
from jax.experimental import pallas as pl
from jax.experimental.pallas import tpu as pltpu
from jax.experimental.pallas import tpu_sc as plsc

N = 320000
D = 128
G = 1024
EPS = 1e-5
NEG = -3.0e38

NC = 2
NS = 16
NW = NC * NS
RPW = N // NW
CHUNK = 400
NCHUNK = RPW // CHUNK
NBLK = CHUNK // 16
NL = D // 16


def _scalar_store(ref, idx, val):
  idx = jnp.asarray(idx, jnp.int32)
  base = (idx // 16) * 16
  lane = idx - base
  lanes = lax.broadcasted_iota(jnp.int32, (16,), 0)
  cur = ref[pl.ds(base, 16)]
  ref[pl.ds(base, 16)] = jnp.where(lanes == lane, jnp.full((16,), val), cur)


def _sc_reduce_build():
  mesh = plsc.VectorSubcoreMesh(core_axis_name="c", subcore_axis_name="s")
  out_type = (
      jax.ShapeDtypeStruct((G, D), jnp.float32),
      jax.ShapeDtypeStruct((G, D), jnp.float32),
      jax.ShapeDtypeStruct((NW, G), jnp.float32),
      jax.ShapeDtypeStruct((NW, 16), jnp.int32),
      jax.ShapeDtypeStruct((NW, 16), jnp.float32),
      jax.ShapeDtypeStruct((NW, 2 * D), jnp.float32),
      jax.ShapeDtypeStruct((NW, 2 * D), jnp.float32),
  )
  scratch = [
      pltpu.VMEM((CHUNK, D), jnp.float32),
      pltpu.VMEM((CHUNK, D), jnp.float32),
      pltpu.VMEM((CHUNK + 16,), jnp.int32),
      pltpu.VMEM((CHUNK + 16,), jnp.int32),
      pltpu.VMEM((D,), jnp.float32),
      pltpu.VMEM((D,), jnp.float32),
      pltpu.VMEM((G,), jnp.float32),
      pltpu.VMEM((2 * D,), jnp.float32),
      pltpu.VMEM((2 * D,), jnp.float32),
      pltpu.VMEM((16,), jnp.float32),
      pltpu.VMEM((16,), jnp.int32),
      pltpu.SemaphoreType.DMA,
      pltpu.SemaphoreType.DMA,
      pltpu.SemaphoreType.DMA,
      pltpu.SemaphoreType.DMA,
  ]

  @functools.partial(pl.kernel, out_type=out_type, mesh=mesh,
                     scratch_types=scratch)
  def sc_reduce(ne, bt, isum, imax, icnt, esid, ecnt, esum, emax,
                rb0, rb1, bb0, bb1, accs, accm, cloc, es, em, ec, eid,
                smr0, smr1, smb0, smb1):
    cid = lax.axis_index("c")
    sid = lax.axis_index("s")
    wid = sid * NC + cid
    row0 = wid * RPW

    def start_chunk(c, rb, bb, smr, smb):
      base = row0 + c * CHUNK
      pltpu.make_async_copy(ne.at[pl.ds(base, CHUNK)], rb, smr).start()
      pltpu.make_async_copy(bt.at[pl.ds(base, CHUNK)],
                            bb.at[pl.ds(0, CHUNK)], smb).start()

    def wait_chunk(rb, bb, smr, smb):
      pltpu.make_async_copy(ne.at[pl.ds(0, CHUNK)], rb, smr).wait()
      pltpu.make_async_copy(bt.at[pl.ds(0, CHUNK)],
                            bb.at[pl.ds(0, CHUNK)], smb).wait()

    def zcnt(i, _):
      cloc[pl.ds(i * 16, 16)] = jnp.zeros((16,), jnp.float32)
      return 0
    lax.fori_loop(0, G // 16, zcnt, 0)
    eid[...] = jnp.full((16,), -1, jnp.int32)
    ec[...] = jnp.zeros((16,), jnp.float32)

    def make_row_body(rb, bb):
      def row_body(i, carry):
        seg, cnt, nfl = carry
        b = bb[pl.ds(i, 16)][0]
        boundary = b != seg
        do_flush = jnp.logical_and(boundary, cnt > 0)

        @pl.when(do_flush)
        def _():
          cf = cnt.astype(jnp.float32)

          @pl.when(nfl == 0)
          def _():
            for j in range(NL):
              es[pl.ds(j * 16, 16)] = accs[pl.ds(j * 16, 16)]
              em[pl.ds(j * 16, 16)] = accm[pl.ds(j * 16, 16)]
            _scalar_store(ec, 0, cf)
            _scalar_store(eid, 0, seg)

          @pl.when(nfl > 0)
          def _():
            pltpu.sync_copy(accs, isum.at[seg])
            pltpu.sync_copy(accm, imax.at[seg])
            _scalar_store(cloc, seg, cf)

        for j in range(NL):
          v = rb[i, pl.ds(j * 16, 16)]
          os_ = accs[pl.ds(j * 16, 16)]
          om_ = accm[pl.ds(j * 16, 16)]
          accs[pl.ds(j * 16, 16)] = jnp.where(boundary, v, os_ + v)
          accm[pl.ds(j * 16, 16)] = jnp.where(boundary, v, jnp.maximum(om_, v))
        newseg = jnp.where(boundary, b, seg)
        newcnt = jnp.where(boundary, 1, cnt + 1)
        newnfl = jnp.where(do_flush, nfl + 1, nfl)
        return (newseg, newcnt, newnfl)
      return row_body

    def make_chunk_proc(rb, bb):
      row_body = make_row_body(rb, bb)

      def block_body(k, carry):
        seg = carry[0]
        bv = bb[pl.ds(k * 16, 16)]
        fast = jnp.logical_and(bv[0] == seg, bv[15] == seg)

        def fast_fn(cr):
          s_ = [accs[pl.ds(j * 16, 16)] for j in range(NL)]
          m_ = [accm[pl.ds(j * 16, 16)] for j in range(NL)]
          for r in range(16):
            for j in range(NL):
              v = rb[k * 16 + r, pl.ds(j * 16, 16)]
              s_[j] = s_[j] + v
              m_[j] = jnp.maximum(m_[j], v)
          for j in range(NL):
            accs[pl.ds(j * 16, 16)] = s_[j]
            accm[pl.ds(j * 16, 16)] = m_[j]
          return (cr[0], cr[1] + 16, cr[2])

        def slow_fn(cr):
          return lax.fori_loop(k * 16, k * 16 + 16, row_body, cr)

        return lax.cond(fast, fast_fn, slow_fn, carry)

      return lambda carry: lax.fori_loop(0, NBLK, block_body, carry)

    proc0 = make_chunk_proc(rb0, bb0)
    proc1 = make_chunk_proc(rb1, bb1)

    def pair_body(i, carry):
      c0 = 2 * i
      start_chunk(c0 + 1, rb1, bb1, smr1, smb1)
      wait_chunk(rb0, bb0, smr0, smb0)
      carry = proc0(carry)
      start_chunk(c0 + 2, rb0, bb0, smr0, smb0)
      wait_chunk(rb1, bb1, smr1, smb1)
      return proc1(carry)

    init = (jnp.int32(-1), jnp.int32(0), jnp.int32(0))
    start_chunk(0, rb0, bb0, smr0, smb0)
    carry = lax.fori_loop(0, (NCHUNK - 1) // 2, pair_body, init)
    wait_chunk(rb0, bb0, smr0, smb0)
    carry = proc0(carry)

    seg, cnt, nfl = carry
    cf = cnt.astype(jnp.float32)

    @pl.when(nfl == 0)
    def _():
      for j in range(NL):
        es[pl.ds(j * 16, 16)] = accs[pl.ds(j * 16, 16)]
        em[pl.ds(j * 16, 16)] = accm[pl.ds(j * 16, 16)]
      _scalar_store(ec, 0, cf)
      _scalar_store(eid, 0, seg)

    @pl.when(nfl > 0)
    def _():
      for j in range(NL):
        es[pl.ds(D + j * 16, 16)] = accs[pl.ds(j * 16, 16)]
        em[pl.ds(D + j * 16, 16)] = accm[pl.ds(j * 16, 16)]
      _scalar_store(ec, 1, cf)
      _scalar_store(eid, 1, seg)

    pltpu.sync_copy(cloc, icnt.at[wid])
    pltpu.sync_copy(eid, esid.at[wid])
    pltpu.sync_copy(ec, ecnt.at[wid])
    pltpu.sync_copy(es, esum.at[wid])
    pltpu.sync_copy(em, emax.at[wid])

  return sc_reduce


_sc_reduce = _sc_reduce_build()


def _tc_head(isum, imax, icnt, esid2, esidv, ecnt2, esum2, emax2,
             ln_g, ln_b, W1, b1, W2, b2, Ws, bs, Wc, bc,
             score_out, cls_out, mx):
  dn = (((0,), (1,)), ((), ()))
  ones_w = jnp.ones((1, NW), jnp.float32)
  cnt_int = lax.dot_general(icnt[...], ones_w, dn,
                            preferred_element_type=jnp.float32)
  valid = cnt_int > 0.0

  mx[pl.ds(0, G), :] = jnp.where(valid, imax[...], NEG)
  mx[pl.ds(G, 8), :] = jnp.full((8, D), NEG, jnp.float32)

  def merge_max(r, _):
    sd = esid2[r]
    tid = jnp.where(sd >= 0, sd, G)
    row = emax2[pl.ds(r, 1), :]
    cur = mx[pl.ds(tid, 1), :]
    mx[pl.ds(tid, 1), :] = jnp.maximum(cur, row)
    return 0
  lax.fori_loop(0, 2 * NW, merge_max, 0)

  ids = jax.lax.broadcasted_iota(jnp.int32, (2 * NW, G), 1)
  oh = (esidv[...] == ids).astype(jnp.float32)
  dn0 = (((0,), (0,)), ((), ()))
  s = jnp.where(valid, isum[...], 0.0)
  s = s + lax.dot_general(oh, esum2[...], dn0,
                          preferred_element_type=jnp.float32)
  cnt = cnt_int + lax.dot_general(oh, ecnt2[...], dn0,
                                  preferred_element_type=jnp.float32)

  mean = s / jnp.maximum(cnt, 1.0)
  mfin = jnp.where(cnt > 0.0, mx[pl.ds(0, G), :], 0.0)

  g = jnp.concatenate([s, mean, mfin], axis=1)
  mu = jnp.mean(g, axis=1, keepdims=True)
  var = jnp.mean((g - mu) ** 2, axis=1, keepdims=True)
  h = (g - mu) * jax.lax.rsqrt(var + EPS) * ln_g[...] + ln_b[...]

  h = h @ W1[...] + b1[...]
  h = h * jax.nn.sigmoid(h)
  h = h @ W2[...] + b2[...]
  h = h * jax.nn.sigmoid(h)
  score_out[...] = jax.nn.sigmoid(h @ Ws[...] + bs[...])
  cls_out[...] = h @ Wc[...] + bc[...]


@jax.jit
def kernel(node_emb, batch, ln_g, ln_b, W1, b1, W2, b2, Ws, bs, Wc, bc):
  isum, imax, icnt, esid, ecnt, esum, emax = _sc_reduce(node_emb, batch)

  esid2 = esid[:, :2].reshape(2 * NW)
  ecnt2 = ecnt[:, :2].reshape(2 * NW, 1)
  esum2 = esum.reshape(2 * NW, D)
  emax2 = emax.reshape(2 * NW, D)

  vspec = pl.BlockSpec(memory_space=pltpu.VMEM)
  sspec = pl.BlockSpec(memory_space=pltpu.SMEM)
  score, cls = pl.pallas_call(
      _tc_head,
      out_shape=[jax.ShapeDtypeStruct((G, 1), jnp.float32),
                 jax.ShapeDtypeStruct((G, 4), jnp.float32)],
      in_specs=[vspec, vspec, vspec, sspec] + [vspec] * 14,
      out_specs=[vspec, vspec],
      scratch_shapes=[pltpu.VMEM((G + 8, D), jnp.float32)],
  )(isum, imax, icnt, esid2, esid2.reshape(2 * NW, 1), ecnt2, esum2, emax2,
    ln_g.reshape(1, 3 * D), ln_b.reshape(1, 3 * D), W1, b1.reshape(1, D),
    W2, b2.reshape(1, D // 2), Ws, bs.reshape(1, 1), Wc, bc.reshape(1, 4))
  return score[:, 0], cls


# speedup vs baseline: 14.3969x; 1.0526x over previous
import functools

import jax
import jax.numpy as jnp
from jax import lax
from jax.experimental import pallas as pl
from jax.experimental.pallas import tpu as pltpu
from jax.experimental.pallas import tpu_sc as plsc

N = 320000
D = 128
G = 1024
EPS = 1e-5
NEG = -3.0e38

NC = 2
NS = 16
NW = NC * NS
RPW = N // NW
CHUNK = 400
NCHUNK = RPW // CHUNK
NBLK = CHUNK // 16
NL = D // 16


def _scalar_store(ref, idx, val):
  idx = jnp.asarray(idx, jnp.int32)
  base = (idx // 16) * 16
  lane = idx - base
  lanes = lax.broadcasted_iota(jnp.int32, (16,), 0)
  cur = ref[pl.ds(base, 16)]
  ref[pl.ds(base, 16)] = jnp.where(lanes == lane, jnp.full((16,), val), cur)


def _sc_reduce_build():
  mesh = plsc.VectorSubcoreMesh(core_axis_name="c", subcore_axis_name="s")
  out_type = (
      jax.ShapeDtypeStruct((G, 2 * D), jnp.float32),
      jax.ShapeDtypeStruct((NW, G), jnp.float32),
      jax.ShapeDtypeStruct((NW, 16), jnp.int32),
      jax.ShapeDtypeStruct((NW, 16), jnp.float32),
      jax.ShapeDtypeStruct((NW, 4 * D), jnp.float32),
  )
  scratch = [
      pltpu.VMEM((CHUNK, D), jnp.float32),
      pltpu.VMEM((CHUNK, D), jnp.float32),
      pltpu.VMEM((RPW + 16,), jnp.int32),
      pltpu.VMEM((2, 2 * D), jnp.float32),
      pltpu.VMEM((G,), jnp.float32),
      pltpu.VMEM((4 * D,), jnp.float32),
      pltpu.VMEM((16,), jnp.float32),
      pltpu.VMEM((16,), jnp.int32),
      pltpu.SemaphoreType.DMA,
      pltpu.SemaphoreType.DMA,
      pltpu.SemaphoreType.DMA,
      pltpu.SemaphoreType.DMA,
  ]

  @functools.partial(pl.kernel, out_type=out_type, mesh=mesh,
                     scratch_types=scratch)
  def sc_reduce(ne, bt, ism, icnt, esid, ecnt, esm,
                rb0, rb1, bb, acc, cloc, esl, ec, eid,
                smr0, smr1, smb, smf):
    cid = lax.axis_index("c")
    sid = lax.axis_index("s")
    wid = sid * NC + cid
    row0 = wid * RPW

    def start_rows(c, rb, smr):
      pltpu.make_async_copy(ne.at[pl.ds(row0 + c * CHUNK, CHUNK)], rb,
                            smr).start()

    def wait_rows(rb, smr):
      pltpu.make_async_copy(ne.at[pl.ds(0, CHUNK)], rb, smr).wait()

    def wait_flush():
      pltpu.make_async_copy(acc.at[0], ism.at[0], smf).wait()

    pltpu.make_async_copy(bt.at[pl.ds(row0, RPW)],
                          bb.at[pl.ds(0, RPW)], smb).start()
    start_rows(0, rb0, smr0)

    def zcnt(i, _):
      cloc[pl.ds(i * 16, 16)] = jnp.zeros((16,), jnp.float32)
      return 0
    lax.fori_loop(0, G // 16, zcnt, 0)
    eid[...] = jnp.full((16,), -1, jnp.int32)
    ec[...] = jnp.zeros((16,), jnp.float32)

    pltpu.make_async_copy(bt.at[pl.ds(0, RPW)],
                          bb.at[pl.ds(0, RPW)], smb).wait()

    def save_edge(slot, seg, cnt, nfl):
      p = nfl & 1
      for j in range(NL):
        esl[pl.ds(slot * 2 * D + j * 16, 16)] = acc[p, pl.ds(j * 16, 16)]
        esl[pl.ds(slot * 2 * D + D + j * 16, 16)] = (
            acc[p, pl.ds(D + j * 16, 16)])
      _scalar_store(ec, slot, cnt.astype(jnp.float32))
      _scalar_store(eid, slot, seg)

    def make_row_body(rb):
      def row_body(args):
        i, ci, carry = args
        seg, cnt, nfl = carry
        b = bb[pl.ds(i, 16)][0]
        boundary = b != seg
        do_flush = jnp.logical_and(boundary, cnt > 0)

        @pl.when(do_flush)
        def _():
          p = nfl & 1

          @pl.when(nfl == 0)
          def _():
            save_edge(0, seg, cnt, nfl)

          @pl.when(nfl > 0)
          def _():
            @pl.when(nfl > 1)
            def _():
              wait_flush()
            pltpu.make_async_copy(acc.at[p], ism.at[seg], smf).start()
            _scalar_store(cloc, seg, cnt.astype(jnp.float32))

        newseg = jnp.where(boundary, b, seg)
        newcnt = jnp.where(boundary, 1, cnt + 1)
        newnfl = jnp.where(do_flush, nfl + 1, nfl)
        q = newnfl & 1
        for j in range(NL):
          v = rb[ci, pl.ds(j * 16, 16)]
          os_ = acc[q, pl.ds(j * 16, 16)]
          om_ = acc[q, pl.ds(D + j * 16, 16)]
          acc[q, pl.ds(j * 16, 16)] = jnp.where(boundary, v, os_ + v)
          acc[q, pl.ds(D + j * 16, 16)] = jnp.where(
              boundary, v, jnp.maximum(om_, v))
        return (newseg, newcnt, newnfl)
      return row_body

    def make_chunk_proc(rb):
      row_body = make_row_body(rb)

      def proc(carry, cbase):
        def block_body(k, cr):
          seg = cr[0]
          bv = bb[pl.ds(cbase + k * 16, 16)]
          fast = jnp.logical_and(bv[0] == seg, bv[15] == seg)

          def fast_fn(c2):
            p = c2[2] & 1
            s_ = [acc[p, pl.ds(j * 16, 16)] for j in range(NL)]
            m_ = [acc[p, pl.ds(D + j * 16, 16)] for j in range(NL)]
            for r in range(16):
              for j in range(NL):
                v = rb[k * 16 + r, pl.ds(j * 16, 16)]
                s_[j] = s_[j] + v
                m_[j] = jnp.maximum(m_[j], v)
            for j in range(NL):
              acc[p, pl.ds(j * 16, 16)] = s_[j]
              acc[p, pl.ds(D + j * 16, 16)] = m_[j]
            return (c2[0], c2[1] + 16, c2[2])

          def slow_fn(c2):
            def rb_body(r, c3):
              return row_body((cbase + k * 16 + r, k * 16 + r, c3))
            return lax.fori_loop(0, 16, rb_body, c2)

          return lax.cond(fast, fast_fn, slow_fn, cr)

        return lax.fori_loop(0, NBLK, block_body, carry)
      return proc

    proc0 = make_chunk_proc(rb0)
    proc1 = make_chunk_proc(rb1)

    def pair_body(i, carry):
      c0 = 2 * i
      start_rows(c0 + 1, rb1, smr1)
      wait_rows(rb0, smr0)
      carry = proc0(carry, c0 * CHUNK)
      start_rows(c0 + 2, rb0, smr0)
      wait_rows(rb1, smr1)
      return proc1(carry, (c0 + 1) * CHUNK)

    init = (jnp.int32(-1), jnp.int32(0), jnp.int32(0))
    carry = lax.fori_loop(0, (NCHUNK - 1) // 2, pair_body, init)
    wait_rows(rb0, smr0)
    carry = proc0(carry, (NCHUNK - 1) * CHUNK)

    seg, cnt, nfl = carry

    @pl.when(nfl == 0)
    def _():
      save_edge(0, seg, cnt, nfl)

    @pl.when(nfl > 0)
    def _():
      save_edge(1, seg, cnt, nfl)

    @pl.when(nfl >= 2)
    def _():
      wait_flush()

    pltpu.sync_copy(cloc, icnt.at[wid])
    pltpu.sync_copy(eid, esid.at[wid])
    pltpu.sync_copy(ec, ecnt.at[wid])
    pltpu.sync_copy(esl, esm.at[wid])

  return sc_reduce


_sc_reduce = _sc_reduce_build()


def _tc_head(ism, icnt, esid, esidv, ecntv, esm,
             ln_g, ln_b, W1, b1, W2, b2, Ws, bs, Wc, bc,
             score_out, cls_out, mx):
  dn = (((0,), (1,)), ((), ()))
  ones_w = jnp.ones((1, NW), jnp.float32)
  cnt_int = lax.dot_general(icnt[...], ones_w, dn,
                            preferred_element_type=jnp.float32)
  valid = cnt_int > 0.0

  isum = ism[:, pl.ds(0, D)]
  imax = ism[:, pl.ds(D, D)]

  mx[pl.ds(0, G), :] = jnp.where(valid, imax, NEG)
  mx[pl.ds(G, 8), :] = jnp.full((8, D), NEG, jnp.float32)

  def merge_max(w, _):
    for slot in range(2):
      sd = esid[w, slot]
      tid = jnp.where(sd >= 0, sd, G)
      rowf = esm[pl.ds(w, 1), :]
      row = lax.slice(rowf, (0, (2 * slot + 1) * D), (1, (2 * slot + 2) * D))
      cur = mx[pl.ds(tid, 1), :]
      mx[pl.ds(tid, 1), :] = jnp.maximum(cur, row)
    return 0
  lax.fori_loop(0, NW, merge_max, 0)

  ids = jax.lax.broadcasted_iota(jnp.int32, (NW, G), 1)
  oh0 = (esidv[:, pl.ds(0, 1)] == ids).astype(jnp.float32)
  oh1 = (esidv[:, pl.ds(1, 1)] == ids).astype(jnp.float32)
  dn0 = (((0,), (0,)), ((), ()))
  s = jnp.where(valid, isum, 0.0)
  s = s + lax.dot_general(oh0, esm[:, pl.ds(0, D)], dn0,
                          preferred_element_type=jnp.float32)
  s = s + lax.dot_general(oh1, esm[:, pl.ds(2 * D, D)], dn0,
                          preferred_element_type=jnp.float32)
  cnt = cnt_int
  cnt = cnt + lax.dot_general(oh0, ecntv[:, pl.ds(0, 1)], dn0,
                              preferred_element_type=jnp.float32)
  cnt = cnt + lax.dot_general(oh1, ecntv[:, pl.ds(1, 1)], dn0,
                              preferred_element_type=jnp.float32)

  mean = s / jnp.maximum(cnt, 1.0)
  mfin = jnp.where(cnt > 0.0, mx[pl.ds(0, G), :], 0.0)

  g = jnp.concatenate([s, mean, mfin], axis=1)
  mu = jnp.mean(g, axis=1, keepdims=True)
  var = jnp.mean((g - mu) ** 2, axis=1, keepdims=True)
  h = (g - mu) * jax.lax.rsqrt(var + EPS) * ln_g[...] + ln_b[...]

  h = h @ W1[...] + b1[...]
  h = h * jax.nn.sigmoid(h)
  h = h @ W2[...] + b2[...]
  h = h * jax.nn.sigmoid(h)
  score_out[...] = jax.nn.sigmoid(h @ Ws[...] + bs[...])
  cls_out[...] = h @ Wc[...] + bc[...]


@jax.jit
def kernel(node_emb, batch, ln_g, ln_b, W1, b1, W2, b2, Ws, bs, Wc, bc):
  ism, icnt, esid, ecnt, esm = _sc_reduce(node_emb, batch)

  vspec = pl.BlockSpec(memory_space=pltpu.VMEM)
  sspec = pl.BlockSpec(memory_space=pltpu.SMEM)
  score, cls = pl.pallas_call(
      _tc_head,
      out_shape=[jax.ShapeDtypeStruct((G, 1), jnp.float32),
                 jax.ShapeDtypeStruct((G, 4), jnp.float32)],
      in_specs=[vspec, vspec, sspec] + [vspec] * 13,
      out_specs=[vspec, vspec],
      scratch_shapes=[pltpu.VMEM((G + 8, D), jnp.float32)],
  )(ism, icnt, esid, esid, ecnt, esm,
    ln_g.reshape(1, 3 * D), ln_b.reshape(1, 3 * D), W1, b1.reshape(1, D),
    W2, b2.reshape(1, D // 2), Ws, bs.reshape(1, 1), Wc, bc.reshape(1, 4))
  return score[:, 0], cls
